# baseline (speedup 1.0000x reference)
import jax
import jax.numpy as jnp
from jax.experimental import pallas as pl
from jax.experimental.pallas import tpu as pltpu

_K = 6
_NH = 250


def _add_kernel(x_hbm, e_ref, o_hbm, in_buf, out_buf, in_sem, out_sem):
    B = x_hbm.shape[0]
    C = 2 * B

    def in_copy(c):
        s = c % _K
        b, h = divmod(c, 2)
        return pltpu.make_async_copy(
            x_hbm.at[b, pl.ds(h * _NH, _NH)], in_buf.at[s], in_sem.at[s])

    def out_copy(c):
        s = c % _K
        b, h = divmod(c, 2)
        return pltpu.make_async_copy(
            out_buf.at[s], o_hbm.at[b, pl.ds(h * _NH, _NH)], out_sem.at[s])

    for c in range(_K):
        in_copy(c).start()

    for c in range(C):
        s = c % _K
        h = c % 2
        in_copy(c).wait()
        if c >= _K:
            out_copy(c - _K).wait()
        e = e_ref[h * _NH:(h + 1) * _NH, :][:, None, :]
        out_buf[s] = in_buf[s] + e
        out_copy(c).start()
        if c + _K < C:
            in_copy(c + _K).start()

    for c in range(C - _K, C):
        out_copy(c).wait()


def kernel(x, emb_weight):
    B, N, T, D = x.shape
    return pl.pallas_call(
        _add_kernel,
        in_specs=[
            pl.BlockSpec(memory_space=pl.ANY),
            pl.BlockSpec(memory_space=pltpu.VMEM),
        ],
        out_specs=pl.BlockSpec(memory_space=pl.ANY),
        out_shape=jax.ShapeDtypeStruct((B, N, T, D), x.dtype),
        scratch_shapes=[
            pltpu.VMEM((_K, _NH, T, D), x.dtype),
            pltpu.VMEM((_K, _NH, T, D), x.dtype),
            pltpu.SemaphoreType.DMA((_K,)),
            pltpu.SemaphoreType.DMA((_K,)),
        ],
        compiler_params=pltpu.CompilerParams(
            vmem_limit_bytes=60 * 1024 * 1024,
        ),
    )(x, emb_weight)
